# baseline (speedup 1.0000x reference)
import functools

import jax
import jax.numpy as jnp
from jax import lax
from jax.experimental import pallas as pl
from jax.experimental.pallas import tpu as pltpu
from jax.experimental.pallas import tpu_sc as plsc

N_USERS = 100000
N_ITEMS = 100000
N_GENRES = 32
K = 64
BATCH = 4096

NC, NS, L = 2, 16, 16
NW = NC * NS
B_PER_W = BATCH // NW
WORDS_PER_W = B_PER_W * K
FIRE = 8

_Q_STRIDE = N_ITEMS + N_GENRES


def _sc_gather_body(p_hbm, q_hbm, u_hbm, i_hbm, pu_hbm, qi_hbm,
                    u_v, i_v, pidx_v, qidx_v, pout_v, qout_v, psem, qsem):
    wid = lax.axis_index("s") * NC + lax.axis_index("c")
    base = wid * B_PER_W
    pltpu.sync_copy(u_hbm.at[pl.ds(base, B_PER_W)], u_v)
    pltpu.sync_copy(i_hbm.at[pl.ds(base, B_PER_W)], i_v)

    nvec = B_PER_W // L
    u0 = tuple(u_v[pl.ds(v * L, L)] for v in range(nvec))
    i0 = tuple(i_v[pl.ds(v * L, L)] for v in range(nvec))

    def build(k, carry):
        pvs, qvs = carry
        for v in range(nvec):
            pidx_v[k, pl.ds(v * L, L)] = pvs[v]
            qidx_v[k, pl.ds(v * L, L)] = qvs[v]
        return (tuple(x + N_USERS for x in pvs),
                tuple(x + _Q_STRIDE for x in qvs))

    lax.fori_loop(0, K, build, (u0, i0))

    def fire_group(g, carry):
        copies = []
        for j in range(FIRE):
            k = g * FIRE + j
            dst = pl.ds(k * B_PER_W, B_PER_W)
            copies.append(pltpu.async_copy(p_hbm.at[pidx_v.at[k]],
                                           pout_v.at[dst], psem))
            copies.append(pltpu.async_copy(q_hbm.at[qidx_v.at[k]],
                                           qout_v.at[dst], qsem))
        for cp in copies:
            cp.wait()
        return carry

    lax.fori_loop(0, K // FIRE, fire_group, 0)

    pltpu.sync_copy(pout_v, pu_hbm.at[pl.ds(wid * WORDS_PER_W, WORDS_PER_W)])
    pltpu.sync_copy(qout_v, qi_hbm.at[pl.ds(wid * WORDS_PER_W, WORDS_PER_W)])


@functools.cache
def _sc_gather():
  return pl.kernel(
    _sc_gather_body,
    out_type=(jax.ShapeDtypeStruct((NW * WORDS_PER_W,), jnp.float32),
              jax.ShapeDtypeStruct((NW * WORDS_PER_W,), jnp.float32)),
    mesh=plsc.VectorSubcoreMesh(core_axis_name="c", subcore_axis_name="s",
                                num_cores=NC, num_subcores=NS),
    scratch_types=[
        pltpu.VMEM((B_PER_W,), jnp.int32),
        pltpu.VMEM((B_PER_W,), jnp.int32),
        pltpu.VMEM((K, B_PER_W), jnp.int32),
        pltpu.VMEM((K, B_PER_W), jnp.int32),
        pltpu.VMEM((WORDS_PER_W,), jnp.float32),
        pltpu.VMEM((WORDS_PER_W,), jnp.float32),
        pltpu.SemaphoreType.DMA,
        pltpu.SemaphoreType.DMA,
    ],
  )


def _dot(a, b, dims):
    return lax.dot_general(a, b, (dims, ((), ())),
                           preferred_element_type=jnp.float32)


def _mlp_body(pu_ref, qi_ref, g_ref, w1_ref, b1_ref, w2_ref, b2_ref,
              w3_ref, b3_ref, hw_ref, qg_ref, o_ref):
    pu_t = pu_ref[0]
    qi_t = qi_ref[0] + _dot(qg_ref[...], g_ref[...], ((1,), (1,)))
    x_t = jnp.concatenate([pu_t, qi_t], axis=0)
    z1 = _dot(w1_ref[...], x_t, ((1,), (0,))) + b1_ref[...]
    x1 = jnp.maximum(z1, 0.0)
    x2 = jnp.maximum(_dot(w2_ref[...], x1, ((1,), (0,))) + b2_ref[...], 0.0)
    x3 = jnp.maximum(_dot(w3_ref[...], x2, ((1,), (0,))) + b3_ref[...], 0.0)
    o_ref[0] = _dot(hw_ref[...], x3, ((1,), (0,)))


def _mlp_call(pu_sc, qi_sc, genres, W1, b1, W2, b2, W3, b3, h_w, qg):
    full = lambda shape: pl.BlockSpec(shape, lambda w: (0,) * len(shape))
    return pl.pallas_call(
        _mlp_body,
        grid=(NW,),
        in_specs=[
            pl.BlockSpec((1, K, B_PER_W), lambda w: (w, 0, 0)),
            pl.BlockSpec((1, K, B_PER_W), lambda w: (w, 0, 0)),
            pl.BlockSpec((B_PER_W, N_GENRES), lambda w: (w, 0)),
            full(W1.shape), full((256, 1)),
            full(W2.shape), full((128, 1)),
            full(W3.shape), full((64, 1)),
            full(h_w.shape), full(qg.shape),
        ],
        out_specs=pl.BlockSpec((1, 1, B_PER_W), lambda w: (w, 0, 0)),
        out_shape=jax.ShapeDtypeStruct((NW, 1, B_PER_W), jnp.float32),
    )(pu_sc, qi_sc, genres, W1, b1.reshape(-1, 1), W2, b2.reshape(-1, 1),
      W3, b3.reshape(-1, 1), h_w, qg)


def kernel(user_ids, item_ids, genres_one_hot, P_w, Q_w, W1, b1, W2, b2, W3, b3, h_w):
    p_flat = P_w.reshape(-1)
    q_flat = Q_w.reshape(-1)
    pu_flat, qi_flat = _sc_gather()(p_flat, q_flat,
                                    user_ids.astype(jnp.int32),
                                    item_ids.astype(jnp.int32))
    pu_sc = pu_flat.reshape(NW, K, B_PER_W)
    qi_sc = qi_flat.reshape(NW, K, B_PER_W)
    qg = Q_w[:, N_ITEMS:]
    out = _mlp_call(pu_sc, qi_sc, genres_one_hot, W1, b1, W2, b2, W3, b3,
                    h_w, qg)
    return out.reshape(BATCH, 1)


# speedup vs baseline: 1.0548x; 1.0548x over previous
import functools

import jax
import jax.numpy as jnp
from jax import lax
from jax.experimental import pallas as pl
from jax.experimental.pallas import tpu as pltpu
from jax.experimental.pallas import tpu_sc as plsc

N_USERS = 100000
N_ITEMS = 100000
N_GENRES = 32
K = 64
BATCH = 4096

NC, NS, L = 2, 16, 16
NW = NC * NS
B_PER_W = BATCH // NW


def _sc_gather_body(comb_hbm, u_hbm, i_hbm, urows_hbm, irows_hbm,
                    u_v, i_v, urows_v, irows_v, usem, isem):
    wid = lax.axis_index("s") * NC + lax.axis_index("c")
    base = wid * B_PER_W
    pltpu.sync_copy(u_hbm.at[pl.ds(base, B_PER_W)], u_v)
    pltpu.sync_copy(i_hbm.at[pl.ds(base, B_PER_W)], i_v)
    cu = pltpu.async_copy(comb_hbm.at[u_v], urows_v, usem)
    ci = pltpu.async_copy(comb_hbm.at[i_v], irows_v, isem)
    cu.wait()
    ci.wait()
    pltpu.sync_copy(urows_v, urows_hbm.at[pl.ds(base, B_PER_W)])
    pltpu.sync_copy(irows_v, irows_hbm.at[pl.ds(base, B_PER_W)])


@functools.cache
def _sc_gather():
  return pl.kernel(
    _sc_gather_body,
    out_type=(jax.ShapeDtypeStruct((BATCH, 2 * K), jnp.float32),
              jax.ShapeDtypeStruct((BATCH, 2 * K), jnp.float32)),
    mesh=plsc.VectorSubcoreMesh(core_axis_name="c", subcore_axis_name="s",
                                num_cores=NC, num_subcores=NS),
    scratch_types=[
        pltpu.VMEM((B_PER_W,), jnp.int32),
        pltpu.VMEM((B_PER_W,), jnp.int32),
        pltpu.VMEM((B_PER_W, 2 * K), jnp.float32),
        pltpu.VMEM((B_PER_W, 2 * K), jnp.float32),
        pltpu.SemaphoreType.DMA,
        pltpu.SemaphoreType.DMA,
    ],
  )


BLK_B = 512


def _dot(a, b, dims):
    return lax.dot_general(a, b, (dims, ((), ())),
                           preferred_element_type=jnp.float32)


def _mlp_body(pu_ref, qi_ref, g_ref, w1_ref, b1_ref, w2_ref, b2_ref,
              w3_ref, b3_ref, hw_ref, qg_ref, o_ref):
    pu = pu_ref[...][:, :K]
    qi_item = qi_ref[...][:, K:]
    qi = qi_item + _dot(g_ref[...], qg_ref[...], ((1,), (1,)))
    x = jnp.concatenate([pu, qi], axis=1)
    z1 = _dot(x, w1_ref[...], ((1,), (1,))) + b1_ref[...]
    x1 = jnp.maximum(z1, 0.0)
    x2 = jnp.maximum(_dot(x1, w2_ref[...], ((1,), (1,))) + b2_ref[...], 0.0)
    x3 = jnp.maximum(_dot(x2, w3_ref[...], ((1,), (1,))) + b3_ref[...], 0.0)
    o_ref[...] = _dot(x3, hw_ref[...], ((1,), (1,)))


def _mlp_call(urows, irows, genres, W1, b1, W2, b2, W3, b3, h_w, qg):
    full = lambda shape: pl.BlockSpec(shape, lambda i: (0,) * len(shape))
    return pl.pallas_call(
        _mlp_body,
        grid=(BATCH // BLK_B,),
        in_specs=[
            pl.BlockSpec((BLK_B, 2 * K), lambda i: (i, 0)),
            pl.BlockSpec((BLK_B, 2 * K), lambda i: (i, 0)),
            pl.BlockSpec((BLK_B, N_GENRES), lambda i: (i, 0)),
            full(W1.shape), full((1, 256)),
            full(W2.shape), full((1, 128)),
            full(W3.shape), full((1, 64)),
            full(h_w.shape), full(qg.shape),
        ],
        out_specs=pl.BlockSpec((BLK_B, 1), lambda i: (i, 0)),
        out_shape=jax.ShapeDtypeStruct((BATCH, 1), jnp.float32),
    )(urows, irows, genres, W1, b1.reshape(1, -1), W2, b2.reshape(1, -1),
      W3, b3.reshape(1, -1), h_w, qg)


def kernel(user_ids, item_ids, genres_one_hot, P_w, Q_w, W1, b1, W2, b2, W3, b3, h_w):
    comb = jnp.concatenate([P_w.T, Q_w[:, :N_ITEMS].T], axis=1)
    qg = Q_w[:, N_ITEMS:]
    urows, irows = _sc_gather()(comb,
                                user_ids.astype(jnp.int32),
                                item_ids.astype(jnp.int32))
    return _mlp_call(urows, irows, genres_one_hot, W1, b1, W2, b2, W3, b3,
                     h_w, qg)


# speedup vs baseline: 1.6748x; 1.5878x over previous
import functools

import jax
import jax.numpy as jnp
from jax import lax
from jax.experimental import pallas as pl
from jax.experimental.pallas import tpu as pltpu
from jax.experimental.pallas import tpu_sc as plsc

N_USERS = 100000
N_ITEMS = 100000
N_GENRES = 32
K = 64
BATCH = 4096

NC, NS, L = 2, 16, 16
NW = NC * NS

CW = 1280
NCHUNK = N_USERS // CW
TAIL0 = NCHUNK * CW
TAIL_W = N_USERS - TAIL0
ROUNDS = -(-NCHUNK // NW)
LIST_N = BATCH + L


def _extract_chunk(ids_v, out_hbm, buf_v, rel_v, row_v, stage_v,
                   ssem, c0, width, cps):
    lanes = lax.iota(jnp.int32, L)

    def scan(j, off):
        v = ids_v[pl.ds(j * L, L)]
        rel = v - c0
        m = (rel >= 0) & (rel < width)
        cnt = plsc.all_reduce_population_count(m)[0]
        plsc.store_compressed(rel_v.at[pl.ds(off, L)], rel, mask=m)
        plsc.store_compressed(row_v.at[pl.ds(off, L)], j * L + lanes,
                              mask=m)
        return off + cnt

    nmatch = lax.fori_loop(0, BATCH // L, scan, 0)
    ngroups = (nmatch + L - 1) // L
    rel0 = rel_v[pl.ds(0, L)][0]
    row0 = row_v[pl.ds(0, L)][0]
    rel_v[pl.ds(nmatch, L)] = jnp.zeros((L,), jnp.int32) + rel0
    row_v[pl.ds(nmatch, L)] = jnp.zeros((L,), jnp.int32) + row0
    for cp in cps:
        cp.wait()

    def extract(g, _):
        rel = rel_v[pl.ds(g * L, L)]
        rows = row_v[pl.ds(g * L, L)]
        for k in range(K):
            kv = jnp.zeros((L,), jnp.int32) + k
            vals = plsc.load_gather(buf_v, [kv, rel])
            plsc.store_scatter(stage_v, [lanes, kv], vals)
        pltpu.async_copy(stage_v, out_hbm.at[rows], ssem).wait()
        return _

    lax.fori_loop(0, ngroups, extract, 0)


def _extract_table(tab_hbm, tail_hbm, ids_v, out_hbm, chunk_v, tail_v,
                   rel_v, row_v, stage_v, csem, ssem, wid):
    def rnd(r, _):
        c = wid + r * NW

        @pl.when(c < NCHUNK)
        def _go():
            c0 = pl.multiple_of(c * CW, CW)
            cp = pltpu.async_copy(tab_hbm.at[:, pl.ds(c0, CW)], chunk_v, csem)
            _extract_chunk(ids_v, out_hbm, chunk_v, rel_v, row_v,
                           stage_v, ssem, c0, CW, [cp])

        return _

    lax.fori_loop(0, ROUNDS, rnd, 0)

    @pl.when(wid == NW - 1)
    def _tail():
        cp = pltpu.async_copy(tail_hbm, tail_v, csem)
        _extract_chunk(ids_v, out_hbm, tail_v, rel_v, row_v,
                       stage_v, ssem, TAIL0, TAIL_W, [cp])


def _sc_extract_body(p_hbm, q_hbm, pt_hbm, qt_hbm, u_hbm, i_hbm,
                     pu_hbm, qi_hbm,
                     u_v, i_v, chunk_v, tail_v, rel_v, row_v, stage_v,
                     csem, ssem):
    wid = lax.axis_index("s") * NC + lax.axis_index("c")
    pltpu.sync_copy(u_hbm, u_v)
    pltpu.sync_copy(i_hbm, i_v)
    _extract_table(p_hbm, pt_hbm, u_v, pu_hbm, chunk_v, tail_v,
                   rel_v, row_v, stage_v, csem, ssem, wid)
    _extract_table(q_hbm, qt_hbm, i_v, qi_hbm, chunk_v, tail_v,
                   rel_v, row_v, stage_v, csem, ssem, wid)


@functools.cache
def _sc_extract():
  return pl.kernel(
    _sc_extract_body,
    out_type=(jax.ShapeDtypeStruct((BATCH, 2 * K), jnp.float32),
              jax.ShapeDtypeStruct((BATCH, 2 * K), jnp.float32)),
    mesh=plsc.VectorSubcoreMesh(core_axis_name="c", subcore_axis_name="s",
                                num_cores=NC, num_subcores=NS),
    scratch_types=[
        pltpu.VMEM((BATCH,), jnp.int32),
        pltpu.VMEM((BATCH,), jnp.int32),
        pltpu.VMEM((K, CW), jnp.float32),
        pltpu.VMEM((K, TAIL_W), jnp.float32),
        pltpu.VMEM((LIST_N,), jnp.int32),
        pltpu.VMEM((LIST_N,), jnp.int32),
        pltpu.VMEM((L, 2 * K), jnp.float32),
        pltpu.SemaphoreType.DMA,
        pltpu.SemaphoreType.DMA,
    ],
    compiler_params=pltpu.CompilerParams(needs_layout_passes=False),
  )


BLK_B = 512


def _dot(a, b, dims):
    return lax.dot_general(a, b, (dims, ((), ())),
                           preferred_element_type=jnp.float32)


def _mlp_body(pu_ref, qi_ref, g_ref, w1_ref, b1_ref, w2_ref, b2_ref,
              w3_ref, b3_ref, hw_ref, qg_ref, o_ref):
    pu = pu_ref[...][:, :K]
    qi_item = qi_ref[...][:, :K]
    qi = qi_item + _dot(g_ref[...], qg_ref[...], ((1,), (1,)))
    x = jnp.concatenate([pu, qi], axis=1)
    z1 = _dot(x, w1_ref[...], ((1,), (1,))) + b1_ref[...]
    x1 = jnp.maximum(z1, 0.0)
    x2 = jnp.maximum(_dot(x1, w2_ref[...], ((1,), (1,))) + b2_ref[...], 0.0)
    x3 = jnp.maximum(_dot(x2, w3_ref[...], ((1,), (1,))) + b3_ref[...], 0.0)
    o_ref[...] = _dot(x3, hw_ref[...], ((1,), (1,)))


def _mlp_call(urows, irows, genres, W1, b1, W2, b2, W3, b3, h_w, qg):
    full = lambda shape: pl.BlockSpec(shape, lambda i: (0,) * len(shape))
    return pl.pallas_call(
        _mlp_body,
        grid=(BATCH // BLK_B,),
        in_specs=[
            pl.BlockSpec((BLK_B, 2 * K), lambda i: (i, 0)),
            pl.BlockSpec((BLK_B, 2 * K), lambda i: (i, 0)),
            pl.BlockSpec((BLK_B, N_GENRES), lambda i: (i, 0)),
            full(W1.shape), full((1, 256)),
            full(W2.shape), full((1, 128)),
            full(W3.shape), full((1, 64)),
            full(h_w.shape), full(qg.shape),
        ],
        out_specs=pl.BlockSpec((BLK_B, 1), lambda i: (i, 0)),
        out_shape=jax.ShapeDtypeStruct((BATCH, 1), jnp.float32),
    )(urows, irows, genres, W1, b1.reshape(1, -1), W2, b2.reshape(1, -1),
      W3, b3.reshape(1, -1), h_w, qg)


def kernel(user_ids, item_ids, genres_one_hot, P_w, Q_w, W1, b1, W2, b2, W3, b3, h_w):
    qg = Q_w[:, N_ITEMS:]
    urows, irows = _sc_extract()(P_w, Q_w,
                                 P_w[:, TAIL0:],
                                 Q_w[:, TAIL0:N_ITEMS],
                                 user_ids.astype(jnp.int32),
                                 item_ids.astype(jnp.int32))
    return _mlp_call(urows, irows, genres_one_hot, W1, b1, W2, b2, W3, b3,
                     h_w, qg)


# speedup vs baseline: 1.8197x; 1.0865x over previous
import functools

import jax
import jax.numpy as jnp
from jax import lax
from jax.experimental import pallas as pl
from jax.experimental.pallas import tpu as pltpu
from jax.experimental.pallas import tpu_sc as plsc

N_USERS = 100000
N_ITEMS = 100000
N_GENRES = 32
K = 64
BATCH = 4096

NC, NS, L = 2, 16, 16
NW = NC * NS

CW = 1280
NCHUNK = N_USERS // CW
TAIL0 = NCHUNK * CW
TAIL_W = N_USERS - TAIL0
ROUNDS = -(-NCHUNK // NW)
LIST_N = BATCH + L


def _extract_chunk(ids_v, out_hbm, buf_v, rel_v, row_v, stage_v,
                   ssem, c0, width, cps):
    lanes = lax.iota(jnp.int32, L)

    def scan(j, off):
        v = ids_v[pl.ds(j * L, L)]
        rel = v - c0
        m = (rel >= 0) & (rel < width)
        cnt = plsc.all_reduce_population_count(m)[0]
        plsc.store_compressed(rel_v.at[pl.ds(off, L)], rel, mask=m)
        plsc.store_compressed(row_v.at[pl.ds(off, L)], j * L + lanes,
                              mask=m)
        return off + cnt

    nmatch = lax.fori_loop(0, BATCH // L, scan, 0)
    ngroups = (nmatch + L - 1) // L
    rel0 = rel_v[pl.ds(0, L)][0]
    row0 = row_v[pl.ds(0, L)][0]
    rel_v[pl.ds(nmatch, L)] = jnp.zeros((L,), jnp.int32) + rel0
    row_v[pl.ds(nmatch, L)] = jnp.zeros((L,), jnp.int32) + row0
    for cp in cps:
        cp.wait()

    def extract(g, _):
        rel = rel_v[pl.ds(g * L, L)]
        rows = row_v[pl.ds(g * L, L)]
        for k in range(K):
            kv = jnp.zeros((L,), jnp.int32) + k
            vals = plsc.load_gather(buf_v, [kv, rel])
            plsc.store_scatter(stage_v, [lanes, kv], vals)
        pltpu.async_copy(stage_v, out_hbm.at[rows], ssem).wait()
        return _

    lax.fori_loop(0, ngroups, extract, 0)


def _extract_table(tab_hbm, tail_hbm, ids_v, out_hbm, chunk_v, tail_v,
                   rel_v, row_v, stage_v, csem, ssem, wid, shift, tail_wid):
    base = lax.rem(wid + shift, NW)

    def rnd(r, _):
        c = base + r * NW

        @pl.when(c < NCHUNK)
        def _go():
            c0 = pl.multiple_of(c * CW, CW)
            cp = pltpu.async_copy(tab_hbm.at[:, pl.ds(c0, CW)], chunk_v, csem)
            _extract_chunk(ids_v, out_hbm, chunk_v, rel_v, row_v,
                           stage_v, ssem, c0, CW, [cp])

        return _

    lax.fori_loop(0, ROUNDS, rnd, 0)

    @pl.when(wid == tail_wid)
    def _tail():
        cp = pltpu.async_copy(tail_hbm, tail_v, csem)
        _extract_chunk(ids_v, out_hbm, tail_v, rel_v, row_v,
                       stage_v, ssem, TAIL0, TAIL_W, [cp])


def _sc_extract_body(p_hbm, q_hbm, pt_hbm, qt_hbm, u_hbm, i_hbm,
                     pu_hbm, qi_hbm,
                     u_v, i_v, chunk_v, tail_v, rel_v, row_v, stage_v,
                     csem, ssem):
    wid = lax.axis_index("s") * NC + lax.axis_index("c")
    pltpu.sync_copy(u_hbm, u_v)
    pltpu.sync_copy(i_hbm, i_v)
    _extract_table(p_hbm, pt_hbm, u_v, pu_hbm, chunk_v, tail_v,
                   rel_v, row_v, stage_v, csem, ssem, wid, 0, NW - 1)
    _extract_table(q_hbm, qt_hbm, i_v, qi_hbm, chunk_v, tail_v,
                   rel_v, row_v, stage_v, csem, ssem, wid, NW // 2, NW - 2)


@functools.cache
def _sc_extract():
  return pl.kernel(
    _sc_extract_body,
    out_type=(jax.ShapeDtypeStruct((BATCH, 2 * K), jnp.float32),
              jax.ShapeDtypeStruct((BATCH, 2 * K), jnp.float32)),
    mesh=plsc.VectorSubcoreMesh(core_axis_name="c", subcore_axis_name="s",
                                num_cores=NC, num_subcores=NS),
    scratch_types=[
        pltpu.VMEM((BATCH,), jnp.int32),
        pltpu.VMEM((BATCH,), jnp.int32),
        pltpu.VMEM((K, CW), jnp.float32),
        pltpu.VMEM((K, TAIL_W), jnp.float32),
        pltpu.VMEM((LIST_N,), jnp.int32),
        pltpu.VMEM((LIST_N,), jnp.int32),
        pltpu.VMEM((L, 2 * K), jnp.float32),
        pltpu.SemaphoreType.DMA,
        pltpu.SemaphoreType.DMA,
    ],
    compiler_params=pltpu.CompilerParams(needs_layout_passes=False),
  )


BLK_B = 512


def _dot(a, b, dims):
    return lax.dot_general(a, b, (dims, ((), ())),
                           preferred_element_type=jnp.float32)


def _mlp_body(pu_ref, qi_ref, g_ref, w1_ref, b1_ref, w2_ref, b2_ref,
              w3_ref, b3_ref, hw_ref, qg_ref, o_ref):
    pu = pu_ref[...][:, :K]
    qi_item = qi_ref[...][:, :K]
    qi = qi_item + _dot(g_ref[...], qg_ref[...], ((1,), (1,)))
    x = jnp.concatenate([pu, qi], axis=1)
    z1 = _dot(x, w1_ref[...], ((1,), (1,))) + b1_ref[...]
    x1 = jnp.maximum(z1, 0.0)
    x2 = jnp.maximum(_dot(x1, w2_ref[...], ((1,), (1,))) + b2_ref[...], 0.0)
    x3 = jnp.maximum(_dot(x2, w3_ref[...], ((1,), (1,))) + b3_ref[...], 0.0)
    o_ref[...] = _dot(x3, hw_ref[...], ((1,), (1,)))


def _mlp_call(urows, irows, genres, W1, b1, W2, b2, W3, b3, h_w, qg):
    full = lambda shape: pl.BlockSpec(shape, lambda i: (0,) * len(shape))
    return pl.pallas_call(
        _mlp_body,
        grid=(BATCH // BLK_B,),
        in_specs=[
            pl.BlockSpec((BLK_B, 2 * K), lambda i: (i, 0)),
            pl.BlockSpec((BLK_B, 2 * K), lambda i: (i, 0)),
            pl.BlockSpec((BLK_B, N_GENRES), lambda i: (i, 0)),
            full(W1.shape), full((1, 256)),
            full(W2.shape), full((1, 128)),
            full(W3.shape), full((1, 64)),
            full(h_w.shape), full(qg.shape),
        ],
        out_specs=pl.BlockSpec((BLK_B, 1), lambda i: (i, 0)),
        out_shape=jax.ShapeDtypeStruct((BATCH, 1), jnp.float32),
    )(urows, irows, genres, W1, b1.reshape(1, -1), W2, b2.reshape(1, -1),
      W3, b3.reshape(1, -1), h_w, qg)


def kernel(user_ids, item_ids, genres_one_hot, P_w, Q_w, W1, b1, W2, b2, W3, b3, h_w):
    qg = Q_w[:, N_ITEMS:]
    urows, irows = _sc_extract()(P_w, Q_w,
                                 P_w[:, TAIL0:],
                                 Q_w[:, TAIL0:N_ITEMS],
                                 user_ids.astype(jnp.int32),
                                 item_ids.astype(jnp.int32))
    return _mlp_call(urows, irows, genres_one_hot, W1, b1, W2, b2, W3, b3,
                     h_w, qg)
